# Initial kernel scaffold; baseline (speedup 1.0000x reference)
import jax
import jax.numpy as jnp
from jax.experimental import pallas as pl


def kernel(world_pos, mesh_pos, node_type, cells, params, is_training):
    raise NotImplementedError("write your pallas kernel here")


# speedup vs baseline: 3.1474x; 3.1474x over previous
import functools

import jax
import jax.numpy as jnp
from jax import lax
from jax.experimental import pallas as pl
from jax.experimental.pallas import tpu as pltpu
from jax.experimental.pallas import tpu_sc as plsc

N = 10000
NPAD = 10240
E = 120000
EPAD = 122880
LAT = 128
BE = 512
BN = 512
F32 = jnp.float32
BF16 = jnp.bfloat16

_sc_mesh = plsc.VectorSubcoreMesh(core_axis_name="c", subcore_axis_name="s")


def _full(x):
    nd = x.ndim
    return pl.BlockSpec(x.shape, lambda i, _nd=nd: (0,) * _nd)


def _bdot(a, b):
    return lax.dot_general(a.astype(BF16), b.astype(BF16), (((1,), (0,)), ((), ())),
                           preferred_element_type=F32)


def _ln(x, g, b):
    mu = jnp.mean(x, axis=-1, keepdims=True)
    xc = x - mu
    var = jnp.mean(xc * xc, axis=-1, keepdims=True)
    return xc * lax.rsqrt(var + 1e-5) * g + b


def _sc_gather2(table, idx_s, idx_r):
    d = table.shape[1]
    w = 128
    out_sd = jax.ShapeDtypeStruct((EPAD, d), table.dtype)

    @functools.partial(pl.kernel, out_type=(out_sd, out_sd), mesh=_sc_mesh)
    def k(tab_hbm, is_hbm, ir_hbm, os_hbm, or_hbm):
        def body(is_v, ir_v, os_v, or_v):
            pltpu.sync_copy(tab_hbm.at[is_v.at[0]], os_v)
            pltpu.sync_copy(tab_hbm.at[ir_v.at[0]], or_v)

        pltpu.emit_pipeline(
            body,
            grid=(EPAD // w,),
            in_specs=[pl.BlockSpec((1, w), lambda i: (0, i)),
                      pl.BlockSpec((1, w), lambda i: (0, i))],
            out_specs=[pl.BlockSpec((w, d), lambda i: (i, 0)),
                       pl.BlockSpec((w, d), lambda i: (i, 0))],
            core_axis_name=("c", "s"),
            dimension_semantics=(pltpu.PARALLEL,),
        )(is_hbm, ir_hbm, os_hbm, or_hbm)

    return k(table, idx_s, idx_r)


def _sc_scatter_add(data, idx):
    ch = 128
    per_core = EPAD // 2
    per_sub = per_core // 16
    chunks = per_sub // ch
    rows_sub = NPAD // 16

    @functools.partial(
        pl.kernel,
        out_type=jax.ShapeDtypeStruct((2, NPAD, LAT), F32),
        mesh=_sc_mesh,
        scratch_types=[pltpu.VMEM((ch, LAT), F32),
                       pltpu.VMEM((ch,), jnp.int32),
                       pltpu.VMEM_SHARED((NPAD, LAT), F32)],
    )
    def k(d_hbm, i_hbm, o_hbm, buf, ibuf, acc):
        c = lax.axis_index("c")
        s = lax.axis_index("s")

        @pl.loop(0, ch)
        def _zero_row(r):
            @pl.loop(0, LAT, step=16)
            def _zero_lane(j):
                buf[pl.ds(r, 1), pl.ds(j, 16)] = jnp.zeros((1, 16), F32)

        @pl.loop(0, rows_sub, step=ch)
        def _zero_acc(r0):
            pltpu.sync_copy(buf, acc.at[pl.ds(s * rows_sub + r0, ch), :])

        plsc.subcore_barrier()

        base0 = c * per_core + s * per_sub

        @pl.loop(0, chunks)
        def _scatter(t):
            b = base0 + t * ch
            pltpu.sync_copy(d_hbm.at[pl.ds(b, ch), :], buf)
            pltpu.sync_copy(i_hbm.at[pl.ds(b, ch)], ibuf)
            pltpu.sync_copy(buf, acc.at[ibuf], add=True)

        plsc.subcore_barrier()
        pltpu.sync_copy(acc.at[pl.ds(s * rows_sub, rows_sub), :],
                        o_hbm.at[c, pl.ds(s * rows_sub, rows_sub), :])

    return k(data, idx)


def _hist_kernel(t_ref, out_ref):
    oh = (t_ref[...] == lax.broadcasted_iota(jnp.int32, (BN, 16), 1)).astype(F32)

    @pl.when(pl.program_id(0) == 0)
    def _():
        out_ref[...] = jnp.zeros_like(out_ref)

    out_ref[...] += jnp.sum(oh, axis=0, keepdims=True)


def _type_counts(tpad):
    return pl.pallas_call(
        _hist_kernel,
        grid=(NPAD // BN,),
        in_specs=[pl.BlockSpec((BN, 1), lambda i: (i, 0))],
        out_specs=pl.BlockSpec((1, 16), lambda i: (0, 0)),
        out_shape=jax.ShapeDtypeStruct((1, 16), F32),
    )(tpad)


def _efeat_kernel(gs_ref, gr_ref, m_ref, feat_ref, stat_ref):
    d = gs_ref[...] - gr_ref[...]
    dw = d[:, 0:3]
    dm = d[:, 3:6]
    nw = jnp.sqrt(jnp.sum(dw * dw, axis=1, keepdims=True))
    nm = jnp.sqrt(jnp.sum(dm * dm, axis=1, keepdims=True))
    m = m_ref[...]
    feat = jnp.concatenate([dw, nw, dm, nm, m, jnp.zeros((BE, 7), F32)], axis=1)
    feat_ref[...] = feat
    s0 = jnp.sum(feat * m, axis=0, keepdims=True)
    s1 = jnp.sum(feat * feat * m, axis=0, keepdims=True)

    @pl.when(pl.program_id(0) == 0)
    def _():
        stat_ref[...] = jnp.zeros_like(stat_ref)

    stat_ref[...] += jnp.concatenate([s0, s1, jnp.zeros((6, 16), F32)], axis=0)


def _edge_features(gps, gpr, mask_col):
    return pl.pallas_call(
        _efeat_kernel,
        grid=(EPAD // BE,),
        in_specs=[pl.BlockSpec((BE, 128), lambda i: (i, 0)),
                  pl.BlockSpec((BE, 128), lambda i: (i, 0)),
                  pl.BlockSpec((BE, 1), lambda i: (i, 0))],
        out_specs=[pl.BlockSpec((BE, 16), lambda i: (i, 0)),
                   pl.BlockSpec((8, 16), lambda i: (0, 0))],
        out_shape=[jax.ShapeDtypeStruct((EPAD, 16), F32),
                   jax.ShapeDtypeStruct((8, 16), F32)],
    )(gps, gpr, mask_col)


def _enc_kernel(x_ref, mu_ref, istd_ref, w1_ref, b1_ref, w2_ref, b2_ref,
                w3_ref, b3_ref, g_ref, be_ref, out_ref):
    x = (x_ref[...] - mu_ref[...]) * istd_ref[...]
    h = jnp.maximum(_bdot(x, w1_ref[...]) + b1_ref[...], 0.0)
    h = jnp.maximum(_bdot(h, w2_ref[...]) + b2_ref[...], 0.0)
    h = _bdot(h, w3_ref[...]) + b3_ref[...]
    out_ref[...] = _ln(h, g_ref[...], be_ref[...])


def _encoder(x, blk, ws):
    rows = x.shape[0]
    return pl.pallas_call(
        _enc_kernel,
        grid=(rows // blk,),
        in_specs=[pl.BlockSpec((blk, x.shape[1]), lambda i: (i, 0))] +
                 [_full(w) for w in ws],
        out_specs=pl.BlockSpec((blk, LAT), lambda i: (i, 0)),
        out_shape=jax.ShapeDtypeStruct((rows, LAT), F32),
    )(x, *ws)


def _onehot_kernel(t_ref, out_ref):
    out_ref[...] = (t_ref[...] == lax.broadcasted_iota(jnp.int32, (BN, 16), 1)
                    ).astype(F32)


def _onehot16(tpad):
    return pl.pallas_call(
        _onehot_kernel,
        grid=(NPAD // BN,),
        in_specs=[pl.BlockSpec((BN, 1), lambda i: (i, 0))],
        out_specs=pl.BlockSpec((BN, 16), lambda i: (i, 0)),
        out_shape=jax.ShapeDtypeStruct((NPAD, 16), F32),
    )(tpad)


def _edge_mp_kernel(gs_ref, gr_ref, el_ref, w1s_ref, w1r_ref, w1e_ref, b1_ref,
                    w2_ref, b2_ref, w3_ref, b3_ref, g_ref, be_ref, out_ref):
    el = el_ref[...]
    h = (_bdot(gs_ref[...], w1s_ref[...]) + _bdot(gr_ref[...], w1r_ref[...]) +
         _bdot(el, w1e_ref[...]) + b1_ref[...])
    h = jnp.maximum(h, 0.0)
    h = jnp.maximum(_bdot(h, w2_ref[...]) + b2_ref[...], 0.0)
    h = _bdot(h, w3_ref[...]) + b3_ref[...]
    out_ref[...] = _ln(h, g_ref[...], be_ref[...]) + el


def _edge_mp(gs, gr, el, ws):
    return pl.pallas_call(
        _edge_mp_kernel,
        grid=(EPAD // BE,),
        in_specs=[pl.BlockSpec((BE, LAT), lambda i: (i, 0))] * 3 +
                 [_full(w) for w in ws],
        out_specs=pl.BlockSpec((BE, LAT), lambda i: (i, 0)),
        out_shape=jax.ShapeDtypeStruct((EPAD, LAT), F32),
    )(gs, gr, el, *ws)


def _node_mp_kernel(nl_ref, a0_ref, a1_ref, w1n_ref, w1a_ref, b1_ref,
                    w2_ref, b2_ref, w3_ref, b3_ref, g_ref, be_ref, out_ref):
    nl = nl_ref[...]
    agg = a0_ref[...] + a1_ref[...]
    h = _bdot(nl, w1n_ref[...]) + _bdot(agg, w1a_ref[...]) + b1_ref[...]
    h = jnp.maximum(h, 0.0)
    h = jnp.maximum(_bdot(h, w2_ref[...]) + b2_ref[...], 0.0)
    h = _bdot(h, w3_ref[...]) + b3_ref[...]
    out_ref[...] = _ln(h, g_ref[...], be_ref[...]) + nl


def _node_mp(nl, a0, a1, ws):
    return pl.pallas_call(
        _node_mp_kernel,
        grid=(NPAD // BN,),
        in_specs=[pl.BlockSpec((BN, LAT), lambda i: (i, 0))] * 3 +
                 [_full(w) for w in ws],
        out_specs=pl.BlockSpec((BN, LAT), lambda i: (i, 0)),
        out_shape=jax.ShapeDtypeStruct((NPAD, LAT), F32),
    )(nl, a0, a1, *ws)


def _dec_kernel(x_ref, w1_ref, b1_ref, w2_ref, b2_ref, w3_ref, b3_ref, out_ref):
    h = jnp.maximum(_bdot(x_ref[...], w1_ref[...]) + b1_ref[...], 0.0)
    h = jnp.maximum(_bdot(h, w2_ref[...]) + b2_ref[...], 0.0)
    out_ref[...] = _bdot(h, w3_ref[...]) + b3_ref[...]


def _decoder(x, ws):
    return pl.pallas_call(
        _dec_kernel,
        grid=(NPAD // BN,),
        in_specs=[pl.BlockSpec((BN, LAT), lambda i: (i, 0))] +
                 [_full(w) for w in ws],
        out_specs=pl.BlockSpec((BN, LAT), lambda i: (i, 0)),
        out_shape=jax.ShapeDtypeStruct((NPAD, LAT), F32),
    )(x, *ws)


def _row(b):
    return b.reshape(1, -1)


def kernel(world_pos, mesh_pos, node_type, cells, params, is_training):
    n = world_pos.shape[0]

    e = jnp.concatenate([cells[:, [0, 1]], cells[:, [1, 2]], cells[:, [2, 0]]], axis=0)
    e = jnp.sort(e, axis=1)
    keys = e[:, 0] * n + e[:, 1]
    order = jnp.argsort(keys)
    e = e[order]
    keys = keys[order]
    first = jnp.concatenate([jnp.ones((1,), jnp.bool_), keys[1:] != keys[:-1]])
    senders = jnp.concatenate([e[:, 0], e[:, 1]]).astype(jnp.int32)
    receivers = jnp.concatenate([e[:, 1], e[:, 0]]).astype(jnp.int32)
    mask = jnp.concatenate([first, first]).astype(F32)

    pad = EPAD - E
    senders_p = jnp.pad(senders, (0, pad))
    receivers_p = jnp.pad(receivers, (0, pad))
    mask_p = jnp.pad(mask, (0, pad))
    scat_idx = jnp.where(mask_p > 0, receivers_p, N)
    senders2d = senders_p.reshape(1, EPAD)
    receivers2d = receivers_p.reshape(1, EPAD)
    mask_col = mask_p.reshape(EPAD, 1)

    pos_tab = jnp.zeros((NPAD, 128), F32)
    pos_tab = pos_tab.at[:n, 0:3].set(world_pos).at[:n, 3:6].set(mesh_pos)
    tpad = jnp.full((NPAD, 1), 15, jnp.int32).at[:n].set(node_type.astype(jnp.int32))

    gps, gpr = _sc_gather2(pos_tab, senders2d, receivers2d)
    feat, stats = _edge_features(gps, gpr, mask_col)

    cnt = stats[0, 8]
    emean = stats[0, :8] / cnt
    evar = jnp.clip(stats[1, :8] / cnt - emean * emean, 0.0)
    estd = jnp.maximum(jnp.sqrt(evar), 1e-8)

    counts = _type_counts(tpad)[0, :9]
    nmean = counts / n
    nvar = jnp.clip(nmean - nmean * nmean, 0.0)
    nstd = jnp.maximum(jnp.sqrt(nvar), 1e-8)

    pe = params['edge_encoder']
    (ew1, eb1), (ew2, eb2), (ew3, eb3) = pe['mlp']
    eg, ebeta = pe['ln']
    ew1p = jnp.zeros((16, LAT), F32).at[:8].set(ew1)
    emu16 = jnp.zeros((1, 16), F32).at[0, :8].set(emean)
    eistd16 = jnp.ones((1, 16), F32).at[0, :8].set(1.0 / estd)
    edge_lat = _encoder(feat, BE, [emu16, eistd16, ew1p, _row(eb1),
                                   ew2, _row(eb2), ew3, _row(eb3),
                                   _row(eg), _row(ebeta)])

    pn = params['node_encoder']
    (nw1, nb1), (nw2, nb2), (nw3, nb3) = pn['mlp']
    ng, nbeta = pn['ln']
    nw1p = jnp.zeros((16, LAT), F32).at[:9].set(nw1)
    nmu16 = jnp.zeros((1, 16), F32).at[0, :9].set(nmean)
    nistd16 = jnp.ones((1, 16), F32).at[0, :9].set(1.0 / nstd)
    oh = _onehot16(tpad)
    node_lat = _encoder(oh, BN, [nmu16, nistd16, nw1p, _row(nb1),
                                 nw2, _row(nb2), nw3, _row(nb3),
                                 _row(ng), _row(nbeta)])

    for blk in params['blocks']:
        (w1, b1), (w2, b2), (w3, b3) = blk['edge_mlp']
        g, be = blk['edge_ln']
        gs, gr = _sc_gather2(node_lat, senders2d, receivers2d)
        new_edge = _edge_mp(gs, gr, edge_lat,
                            [w1[:LAT], w1[LAT:2 * LAT], w1[2 * LAT:], _row(b1),
                             w2, _row(b2), w3, _row(b3), _row(g), _row(be)])
        aggp = _sc_scatter_add(new_edge, scat_idx)
        (v1, c1), (v2, c2), (v3, c3) = blk['node_mlp']
        gn, ben = blk['node_ln']
        node_lat = _node_mp(node_lat, aggp[0], aggp[1],
                            [v1[:LAT], v1[LAT:], _row(c1), v2, _row(c2),
                             v3, _row(c3), _row(gn), _row(ben)])
        edge_lat = new_edge

    (dw1, db1), (dw2, db2), (dw3, db3) = params['decoder']['mlp']
    dw3p = jnp.zeros((LAT, LAT), F32).at[:, :3].set(dw3)
    db3p = jnp.zeros((LAT,), F32).at[:3].set(db3)
    dec = _decoder(node_lat, [dw1, _row(db1), dw2, _row(db2), dw3p, _row(db3p)])
    out = dec[:n, :3]
    return jnp.where(jnp.asarray(is_training) != 0, out, world_pos + out)
